# Initial kernel scaffold; baseline (speedup 1.0000x reference)
import functools

import jax
import jax.numpy as jnp
from jax import lax
from jax.experimental import pallas as pl
from jax.experimental.pallas import tpu as pltpu
from jax.experimental.pallas import tpu_sc as plsc

N = 10000
E = 320000
F_IN = 128
NOUT = 16
NIN = 48
G = 128
EPS = 1e-5

NC = 2
NS = 16
NW = NC * NS
CHUNK = 128
CH = 80
EPW = CH * CHUNK
E_PAD = NW * EPW
RPS = 626
N_ACC = NS * RPS
NPW = 320
N_POOL = NW * NPW
GP = 136

BN = 1000
GRID = N // BN


def _proj(z, pp):
    za, zm1, zm2 = z[:, 0:16], z[:, 16:32], z[:, 32:48]
    a = pp[3] * jax.nn.relu(za + pp[0]) + pp[4]
    m = pp[5] * (jax.nn.relu(zm1 + pp[1]) * jax.nn.relu(zm2 + pp[2])) + pp[6]
    return jnp.concatenate([a, m], axis=1)


def _d0_body(x_ref, w_ref, pp_ref, am_ref, hc_ref):
    z = jnp.dot(x_ref[...], w_ref[...], preferred_element_type=jnp.float32)
    am_ref[...] = _proj(z, pp_ref[...])
    hc_ref[...] = z[:, 48:64]


def _finish_c(p, cp):
    return cp[1] * jax.nn.relu(p[0] + p[1] + cp[0]) + cp[2]


def _dmid_body(am_ref, p_ref, cp_ref, w_ref, pp_ref, am_o, hc_o):
    amp = am_ref[...]
    c = _finish_c(p_ref[...], cp_ref[...])
    h = jnp.concatenate([amp[:, 0:16], c, amp[:, 16:32]], axis=1)
    z = jnp.dot(h, w_ref[...], preferred_element_type=jnp.float32)
    am_o[...] = _proj(z, pp_ref[...])
    hc_o[...] = z[:, 48:64]


def _d4_body(am_ref, p_ref, cp_ref, h_o):
    amp = am_ref[...]
    c = _finish_c(p_ref[...], cp_ref[...])
    h_o[...] = jnp.concatenate([amp[:, 0:16], c, amp[:, 16:32]], axis=1)


def _d5_body(b_ref, s_ref, x_ref, w2_ref, out_ref):
    b = b_ref[...]
    iota = lax.broadcasted_iota(jnp.int32, (GRID, BN, G), 2)
    mask = (b[:, 0, :, None] == iota).astype(jnp.float32)
    counts = jnp.sum(mask, axis=(0, 1))
    sums = jnp.sum(s_ref[...][:, :G, :], axis=0)
    maxs = jnp.max(x_ref[...][:, :G, :], axis=0)
    meanp = sums / jnp.maximum(counts, 1.0)[:, None]
    hg = jnp.concatenate([meanp, maxs], axis=1)
    w2b = w2_ref[...]
    logits = jnp.dot(hg, w2b[:96], preferred_element_type=jnp.float32) + w2b[96]
    mx = jnp.max(logits, axis=1, keepdims=True)
    sh = logits - mx
    out_ref[...] = sh - jnp.log(jnp.sum(jnp.exp(sh), axis=1, keepdims=True))


_d0 = pl.pallas_call(
    _d0_body,
    grid=(GRID,),
    in_specs=[
        pl.BlockSpec((BN, F_IN), lambda i: (i, 0)),
        pl.BlockSpec((F_IN, 64), lambda i: (0, 0)),
        pl.BlockSpec((8, 16), lambda i: (0, 0)),
    ],
    out_specs=[
        pl.BlockSpec((BN, 32), lambda i: (i, 0)),
        pl.BlockSpec((BN, 16), lambda i: (i, 0)),
    ],
    out_shape=[
        jax.ShapeDtypeStruct((N, 32), jnp.float32),
        jax.ShapeDtypeStruct((N, 16), jnp.float32),
    ],
)

_dmid = pl.pallas_call(
    _dmid_body,
    grid=(GRID,),
    in_specs=[
        pl.BlockSpec((BN, 32), lambda i: (i, 0)),
        pl.BlockSpec((NC, BN, 16), lambda i: (0, i, 0)),
        pl.BlockSpec((8, 16), lambda i: (0, 0)),
        pl.BlockSpec((NIN, 64), lambda i: (0, 0)),
        pl.BlockSpec((8, 16), lambda i: (0, 0)),
    ],
    out_specs=[
        pl.BlockSpec((BN, 32), lambda i: (i, 0)),
        pl.BlockSpec((BN, 16), lambda i: (i, 0)),
    ],
    out_shape=[
        jax.ShapeDtypeStruct((N, 32), jnp.float32),
        jax.ShapeDtypeStruct((N, 16), jnp.float32),
    ],
)

_d4 = pl.pallas_call(
    _d4_body,
    grid=(GRID,),
    in_specs=[
        pl.BlockSpec((BN, 32), lambda i: (i, 0)),
        pl.BlockSpec((NC, BN, 16), lambda i: (0, i, 0)),
        pl.BlockSpec((8, 16), lambda i: (0, 0)),
    ],
    out_specs=pl.BlockSpec((BN, NIN), lambda i: (i, 0)),
    out_shape=jax.ShapeDtypeStruct((N_POOL, NIN), jnp.float32),
)

_d5 = pl.pallas_call(
    _d5_body,
    grid=(1,),
    in_specs=[
        pl.BlockSpec((GRID, 1, BN), lambda i: (0, 0, 0)),
        pl.BlockSpec((NW, GP, NIN), lambda i: (0, 0, 0)),
        pl.BlockSpec((NW, GP, NIN), lambda i: (0, 0, 0)),
        pl.BlockSpec((104, 6), lambda i: (0, 0)),
    ],
    out_specs=pl.BlockSpec((G, 6), lambda i: (0, 0)),
    out_shape=jax.ShapeDtypeStruct((G, 6), jnp.float32),
)


_MESH = plsc.VectorSubcoreMesh(core_axis_name="c", subcore_axis_name="s")


@functools.partial(
    pl.kernel,
    out_type=jax.ShapeDtypeStruct((NC, N_ACC, NOUT), jnp.float32),
    mesh=_MESH,
    scratch_types=[
        pltpu.VMEM((CH, CHUNK), jnp.int32),
        pltpu.VMEM((CH, CHUNK), jnp.int32),
        pltpu.VMEM((CHUNK, NOUT), jnp.float32),
        pltpu.VMEM((RPS, NOUT), jnp.float32),
        pltpu.VMEM_SHARED((N_ACC, NOUT), jnp.float32),
        pltpu.SemaphoreType.DMA,
    ],
)
def _agg(table_hbm, src_hbm, dst_hbm, out_hbm,
         src_v, dst_v, rows_v, zbuf_v, acc_sh, sem):
    c = lax.axis_index("c")
    s = lax.axis_index("s")
    w = c * NS + s
    pltpu.sync_copy(src_hbm.at[w], src_v)
    pltpu.sync_copy(dst_hbm.at[w], dst_v)

    def zero_row(i, carry):
        zbuf_v[i, :] = jnp.zeros((NOUT,), jnp.float32)
        return carry
    lax.fori_loop(0, RPS, zero_row, 0)
    pltpu.sync_copy(zbuf_v, acc_sh.at[pl.ds(s * RPS, RPS)])
    plsc.subcore_barrier()

    def step(j, carry):
        pltpu.async_copy(table_hbm.at[src_v.at[j]], rows_v, sem).wait()
        pltpu.sync_copy(rows_v, acc_sh.at[dst_v.at[j]], add=True)
        return carry
    lax.fori_loop(0, CH, step, 0)
    plsc.subcore_barrier()
    pltpu.sync_copy(acc_sh.at[pl.ds(s * RPS, RPS)],
                    out_hbm.at[c, pl.ds(s * RPS, RPS)])


_NEG = float("-inf")


@functools.partial(
    pl.kernel,
    out_type=[
        jax.ShapeDtypeStruct((NW, GP, NIN), jnp.float32),
        jax.ShapeDtypeStruct((NW, GP, NIN), jnp.float32),
    ],
    mesh=_MESH,
    scratch_types=[
        pltpu.VMEM((NPW, NIN), jnp.float32),
        pltpu.SMEM((NPW,), jnp.int32),
        pltpu.VMEM((GP, NIN), jnp.float32),
        pltpu.VMEM((GP, NIN), jnp.float32),
    ],
)
def _pool(h_hbm, b_hbm, sum_hbm, max_hbm, h_v, b_s, sum_v, max_v):
    c = lax.axis_index("c")
    s = lax.axis_index("s")
    w = c * NS + s
    pltpu.sync_copy(h_hbm.at[pl.ds(w * NPW, NPW)], h_v)
    pltpu.sync_copy(b_hbm.at[pl.ds(w * NPW, NPW)], b_s)

    def init_row(i, carry):
        for k in range(3):
            sum_v[i, pl.ds(16 * k, 16)] = jnp.zeros((16,), jnp.float32)
            max_v[i, pl.ds(16 * k, 16)] = jnp.full((16,), _NEG, jnp.float32)
        return carry
    lax.fori_loop(0, GP, init_row, 0)

    def scan_node(i, carry):
        cur, s0, s1, s2, m0, m1, m2 = carry
        seg = b_s[i]
        r0 = h_v[i, pl.ds(0, 16)]
        r1 = h_v[i, pl.ds(16, 16)]
        r2 = h_v[i, pl.ds(32, 16)]
        changed = seg != cur

        @pl.when(changed)
        def _flush():
            sum_v[cur, pl.ds(0, 16)] = s0
            sum_v[cur, pl.ds(16, 16)] = s1
            sum_v[cur, pl.ds(32, 16)] = s2
            max_v[cur, pl.ds(0, 16)] = m0
            max_v[cur, pl.ds(16, 16)] = m1
            max_v[cur, pl.ds(32, 16)] = m2

        s0 = jnp.where(changed, r0, s0 + r0)
        s1 = jnp.where(changed, r1, s1 + r1)
        s2 = jnp.where(changed, r2, s2 + r2)
        m0 = jnp.where(changed, r0, jnp.maximum(m0, r0))
        m1 = jnp.where(changed, r1, jnp.maximum(m1, r1))
        m2 = jnp.where(changed, r2, jnp.maximum(m2, r2))
        return (seg, s0, s1, s2, m0, m1, m2)

    z = jnp.zeros((16,), jnp.float32)
    ninf = jnp.full((16,), _NEG, jnp.float32)
    cur, s0, s1, s2, m0, m1, m2 = lax.fori_loop(
        0, NPW, scan_node, (b_s[0], z, z, z, ninf, ninf, ninf))
    sum_v[cur, pl.ds(0, 16)] = s0
    sum_v[cur, pl.ds(16, 16)] = s1
    sum_v[cur, pl.ds(32, 16)] = s2
    max_v[cur, pl.ds(0, 16)] = m0
    max_v[cur, pl.ds(16, 16)] = m1
    max_v[cur, pl.ds(32, 16)] = m2

    pltpu.sync_copy(sum_v, sum_hbm.at[w])
    pltpu.sync_copy(max_v, max_hbm.at[w])


def kernel(x, edge_index, batch,
           Wa0, bWa0, Wc0, bWc0, Wm10, bWm10, Wm20, bWm20, g0, beta0,
           Wa1, bWa1, Wc1, bWc1, Wm11, bWm11, Wm21, bWm21, g1, beta1,
           Wa2, bWa2, Wc2, bWc2, Wm12, bWm12, Wm22, bWm22, g2, beta2,
           Wa3, bWa3, Wc3, bWc3, Wm13, bWm13, Wm23, bWm23, g3, beta3,
           W2, b2):
    inv = 1.0 / jnp.sqrt(jnp.asarray(1.0 + EPS, jnp.float32))
    Ws, pps, cps = [], [], []
    zero16 = jnp.zeros((16,), jnp.float32)
    for (Wa, bWa, Wc, bWc, Wm1, bWm1, Wm2, bWm2, g, beta) in (
            (Wa0, bWa0, Wc0, bWc0, Wm10, bWm10, Wm20, bWm20, g0, beta0),
            (Wa1, bWa1, Wc1, bWc1, Wm11, bWm11, Wm21, bWm21, g1, beta1),
            (Wa2, bWa2, Wc2, bWc2, Wm12, bWm12, Wm22, bWm22, g2, beta2),
            (Wa3, bWa3, Wc3, bWc3, Wm13, bWm13, Wm23, bWm23, g3, beta3)):
        sca = g * inv
        Ws.append(jnp.concatenate([Wa, Wm1, Wm2, Wc], axis=1))
        pps.append(jnp.stack([bWa, bWm1, bWm2, sca[0:16], beta[0:16],
                              sca[32:48], beta[32:48], zero16]))
        cps.append(jnp.stack([bWc, sca[16:32], beta[16:32], zero16,
                              zero16, zero16, zero16, zero16]))

    src = edge_index[0]
    dst = edge_index[1]
    pad = E_PAD - E
    srcp = jnp.concatenate([src, jnp.zeros((pad,), jnp.int32)]
                           ).reshape(NW, CH, CHUNK)
    dstp = jnp.concatenate([dst, jnp.full((pad,), N, jnp.int32)]
                           ).reshape(NW, CH, CHUNK)
    batch_pad = jnp.concatenate(
        [batch, jnp.full((N_POOL - N,), G, jnp.int32)])
    batch3d = batch.reshape(GRID, 1, BN)
    w2b = jnp.zeros((104, 6), jnp.float32).at[:96].set(W2).at[96].set(b2)

    am, hc = _d0(x, Ws[0], pps[0])
    P = _agg(hc, srcp, dstp)
    for i in (1, 2, 3):
        am, hc = _dmid(am, P, cps[i - 1], Ws[i], pps[i])
        P = _agg(hc, srcp, dstp)
    h4 = _d4(am, P, cps[3])
    sums, maxs = _pool(h4, batch_pad)
    return _d5(batch3d, sums, maxs, w2b)


# speedup vs baseline: 5.8222x; 5.8222x over previous
import functools

import jax
import jax.numpy as jnp
from jax import lax
from jax.experimental import pallas as pl
from jax.experimental.pallas import tpu as pltpu
from jax.experimental.pallas import tpu_sc as plsc

N = 10000
E = 320000
F_IN = 128
NOUT = 16
NIN = 48
G = 128
EPS = 1e-5

NC = 2
NS = 16
NW = NC * NS
CHUNK = 128
CH = 80
EPW = CH * CHUNK
E_PAD = NW * EPW
RPS = 632
N_ACC = NS * RPS
NPW = 320
N_POOL = NW * NPW
GP = 136

BN = 1000
GRID = N // BN


def _proj(z, pp):
    za, zm1, zm2 = z[:, 0:16], z[:, 16:32], z[:, 32:48]
    a = pp[3:4] * jax.nn.relu(za + pp[0:1]) + pp[4:5]
    m = pp[5:6] * (jax.nn.relu(zm1 + pp[1:2]) * jax.nn.relu(zm2 + pp[2:3])) + pp[6:7]
    return jnp.concatenate([a, m], axis=1)


def _d0_body(x_ref, w_ref, pp_ref, a_o, m_o, hc_o):
    z = jnp.dot(x_ref[...], w_ref[...], preferred_element_type=jnp.float32)
    am = _proj(z, pp_ref[...])
    a_o[...] = am[:, 0:16]
    m_o[...] = am[:, 16:32]
    hc_o[...] = z[:, 48:64]


def _finish_c(p, cp):
    return cp[1:2] * jax.nn.relu(p[0] + p[1] + cp[0:1]) + cp[2:3]


def _d1_body(a_ref, m_ref, p_ref, cp_ref, w_ref, pp_ref, c_o, a_o, m_o):
    c = _finish_c(p_ref[...], cp_ref[...])
    h = jnp.concatenate([a_ref[...], c, m_ref[...]], axis=1)
    z = jnp.dot(h, w_ref[...], preferred_element_type=jnp.float32)
    am = _proj(z, pp_ref[...])
    c_o[...] = c
    a_o[...] = am[:, 0:16]
    m_o[...] = am[:, 16:32]


def _agg_cat(qa, qc, qm):
    return jnp.concatenate(
        [qa[0] + qa[1], qc[0] + qc[1], qm[0] + qm[1]], axis=1)


def _dq_body(a_ref, m_ref, qa_ref, qc_ref, qm_ref, wc_ref, cp_ref,
             w_ref, pp_ref, c_o, a_o, m_o):
    cp = cp_ref[...]
    agg = _agg_cat(qa_ref[...], qc_ref[...], qm_ref[...])
    cz = jnp.dot(agg, wc_ref[...], preferred_element_type=jnp.float32)
    c = cp[1:2] * jax.nn.relu(cz + cp[0:1]) + cp[2:3]
    h = jnp.concatenate([a_ref[...], c, m_ref[...]], axis=1)
    z = jnp.dot(h, w_ref[...], preferred_element_type=jnp.float32)
    am = _proj(z, pp_ref[...])
    c_o[...] = c
    a_o[...] = am[:, 0:16]
    m_o[...] = am[:, 16:32]


def _d4_body(qa_ref, qc_ref, qm_ref, wc_ref, cp_ref, c_o):
    cp = cp_ref[...]
    agg = _agg_cat(qa_ref[...], qc_ref[...], qm_ref[...])
    cz = jnp.dot(agg, wc_ref[...], preferred_element_type=jnp.float32)
    c_o[...] = cp[1:2] * jax.nn.relu(cz + cp[0:1]) + cp[2:3]


def _d5_body(b_ref, sa_ref, sc_ref, sm_ref, xa_ref, xc_ref, xm_ref,
             w2_ref, out_ref):
    b = b_ref[...]
    iota = lax.broadcasted_iota(jnp.int32, (GRID, G, BN), 1)
    mask = (b == iota).astype(jnp.float32)
    m2 = jnp.sum(mask, axis=0)
    counts = jnp.dot(m2, jnp.ones((BN, 1), jnp.float32),
                     preferred_element_type=jnp.float32)
    sums = jnp.concatenate(
        [jnp.sum(r[...][:, :G, :], axis=0) for r in (sa_ref, sc_ref, sm_ref)],
        axis=1)
    maxs = jnp.concatenate(
        [jnp.max(r[...][:, :G, :], axis=0) for r in (xa_ref, xc_ref, xm_ref)],
        axis=1)
    meanp = sums / jnp.maximum(counts, 1.0)
    hg = jnp.concatenate([meanp, maxs], axis=1)
    w2b = w2_ref[...]
    logits = jnp.dot(hg, w2b[:96], preferred_element_type=jnp.float32) + w2b[96:97]
    mx = jnp.max(logits, axis=1, keepdims=True)
    sh = logits - mx
    out_ref[...] = sh - jnp.log(jnp.sum(jnp.exp(sh), axis=1, keepdims=True))


_T16 = jax.ShapeDtypeStruct((N_POOL, 16), jnp.float32)
_B16 = pl.BlockSpec((BN, 16), lambda i: (i, 0))
_BP = pl.BlockSpec((NC, BN, 16), lambda i: (0, i, 0))
_B8x16 = pl.BlockSpec((8, 16), lambda i: (0, 0))

_d0 = pl.pallas_call(
    _d0_body,
    grid=(GRID,),
    in_specs=[
        pl.BlockSpec((BN, F_IN), lambda i: (i, 0)),
        pl.BlockSpec((F_IN, 64), lambda i: (0, 0)),
        _B8x16,
    ],
    out_specs=[_B16, _B16, _B16],
    out_shape=[_T16, _T16, _T16],
)

_d1 = pl.pallas_call(
    _d1_body,
    grid=(GRID,),
    in_specs=[
        _B16, _B16, _BP, _B8x16,
        pl.BlockSpec((NIN, NIN), lambda i: (0, 0)),
        _B8x16,
    ],
    out_specs=[_B16, _B16, _B16],
    out_shape=[_T16, _T16, _T16],
)

_dq = pl.pallas_call(
    _dq_body,
    grid=(GRID,),
    in_specs=[
        _B16, _B16, _BP, _BP, _BP,
        pl.BlockSpec((NIN, 16), lambda i: (0, 0)),
        _B8x16,
        pl.BlockSpec((NIN, NIN), lambda i: (0, 0)),
        _B8x16,
    ],
    out_specs=[_B16, _B16, _B16],
    out_shape=[_T16, _T16, _T16],
)

_d4 = pl.pallas_call(
    _d4_body,
    grid=(GRID,),
    in_specs=[
        _BP, _BP, _BP,
        pl.BlockSpec((NIN, 16), lambda i: (0, 0)),
        _B8x16,
    ],
    out_specs=_B16,
    out_shape=_T16,
)

_d5 = pl.pallas_call(
    _d5_body,
    grid=(1,),
    in_specs=[pl.BlockSpec((GRID, 1, BN), lambda i: (0, 0, 0))]
    + [pl.BlockSpec((NW, GP, 16), lambda i: (0, 0, 0))] * 6
    + [pl.BlockSpec((104, 6), lambda i: (0, 0))],
    out_specs=pl.BlockSpec((G, 6), lambda i: (0, 0)),
    out_shape=jax.ShapeDtypeStruct((G, 6), jnp.float32),
)


_MESH = plsc.VectorSubcoreMesh(core_axis_name="c", subcore_axis_name="s")


ZR = 79


def _agg_pass(table_hbm, out_hbm, src_hbm, dst_hbm,
              src_v, dst_v, rows_v, zbuf_v, acc_sh, gsem, isem, c, s, w):
    def zero_row(i, carry):
        zbuf_v[i, :] = jnp.zeros((NOUT,), jnp.float32)
        return carry
    lax.fori_loop(0, ZR, zero_row, 0)
    for k in range(RPS // ZR):
        pltpu.sync_copy(zbuf_v, acc_sh.at[pl.ds(s * RPS + k * ZR, ZR)])
    plsc.subcore_barrier()

    def idx_start(j, p, sem):
        pltpu.make_async_copy(src_hbm.at[w, j], src_v.at[p], sem).start()
        pltpu.make_async_copy(dst_hbm.at[w, j], dst_v.at[p], sem).start()

    def idx_wait(p, sem):
        pltpu.make_async_copy(src_hbm.at[w, 0], src_v.at[p], sem).wait()
        pltpu.make_async_copy(dst_hbm.at[w, 0], dst_v.at[p], sem).wait()

    def gather_start(p, sem):
        pltpu.make_async_copy(
            table_hbm.at[src_v.at[p, 0]], rows_v.at[p], sem).start()

    def gather_wait(p, sem):
        pltpu.make_async_copy(
            table_hbm.at[src_v.at[p, 0]], rows_v.at[p], sem).wait()

    idx_start(0, 0, isem[0])
    idx_wait(0, isem[0])
    gather_start(0, gsem[0])
    idx_start(1, 1, isem[1])

    def half_step(j, p):
        q = 1 - p
        idx_wait(q, isem[q])
        gather_start(q, gsem[q])
        gather_wait(p, gsem[p])
        pltpu.sync_copy(rows_v.at[p], acc_sh.at[dst_v.at[p, 0]], add=True)
        nj = j + 2
        nj = jnp.where(nj >= CH, nj - CH, nj)
        idx_start(nj, p, isem[p])

    def pair(jj, carry):
        half_step(2 * jj, 0)
        half_step(2 * jj + 1, 1)
        return carry
    lax.fori_loop(0, CH // 2, pair, 0)
    gather_wait(0, gsem[0])
    idx_wait(1, isem[1])
    plsc.subcore_barrier()
    pltpu.sync_copy(acc_sh.at[pl.ds(s * RPS, RPS)],
                    out_hbm.at[c, pl.ds(s * RPS, RPS)])


_AGG_SCRATCH = [
    pltpu.VMEM((2, 1, CHUNK), jnp.int32),
    pltpu.VMEM((2, 1, CHUNK), jnp.int32),
    pltpu.VMEM((2, CHUNK, NOUT), jnp.float32),
    pltpu.VMEM((ZR, NOUT), jnp.float32),
    pltpu.VMEM_SHARED((N_ACC, NOUT), jnp.float32),
    pltpu.SemaphoreType.DMA,
    pltpu.SemaphoreType.DMA,
    pltpu.SemaphoreType.DMA,
    pltpu.SemaphoreType.DMA,
]


@functools.partial(
    pl.kernel,
    out_type=jax.ShapeDtypeStruct((NC, N_ACC, NOUT), jnp.float32),
    mesh=_MESH,
    scratch_types=list(_AGG_SCRATCH),
    compiler_params=pltpu.CompilerParams(use_tc_tiling_on_sc=False),
)
def _agg(table_hbm, src_hbm, dst_hbm, out_hbm,
         src_v, dst_v, rows_v, zbuf_v, acc_sh, g0, g1, i0, i1):
    c = lax.axis_index("c")
    s = lax.axis_index("s")
    w = c * NS + s
    _agg_pass(table_hbm, out_hbm, src_hbm, dst_hbm,
              src_v, dst_v, rows_v, zbuf_v, acc_sh, (g0, g1), (i0, i1),
              c, s, w)


@functools.partial(
    pl.kernel,
    out_type=[jax.ShapeDtypeStruct((NC, N_ACC, NOUT), jnp.float32)] * 3,
    mesh=_MESH,
    scratch_types=list(_AGG_SCRATCH),
    compiler_params=pltpu.CompilerParams(use_tc_tiling_on_sc=False),
)
def _agg3(ta_hbm, tc_hbm, tm_hbm, src_hbm, dst_hbm,
          oa_hbm, oc_hbm, om_hbm,
          src_v, dst_v, rows_v, zbuf_v, acc_sh, g0, g1, i0, i1):
    c = lax.axis_index("c")
    s = lax.axis_index("s")
    w = c * NS + s
    for table_hbm, out_hbm in ((ta_hbm, oa_hbm), (tc_hbm, oc_hbm),
                               (tm_hbm, om_hbm)):
        _agg_pass(table_hbm, out_hbm, src_hbm, dst_hbm,
                  src_v, dst_v, rows_v, zbuf_v, acc_sh, (g0, g1), (i0, i1),
                  c, s, w)


_NEG = float("-inf")


@functools.partial(
    pl.kernel,
    out_type=[jax.ShapeDtypeStruct((NW, GP, 16), jnp.float32)] * 6,
    mesh=_MESH,
    scratch_types=[
        pltpu.VMEM((NPW, 16), jnp.float32),
        pltpu.VMEM((NPW, 16), jnp.float32),
        pltpu.VMEM((NPW, 16), jnp.float32),
        pltpu.VMEM((NPW // 16, 16), jnp.int32),
        pltpu.VMEM((GP, 16), jnp.float32),
        pltpu.VMEM((GP, 16), jnp.float32),
        pltpu.VMEM((GP, 16), jnp.float32),
        pltpu.VMEM((GP, 16), jnp.float32),
        pltpu.VMEM((GP, 16), jnp.float32),
        pltpu.VMEM((GP, 16), jnp.float32),
    ],
    compiler_params=pltpu.CompilerParams(
        needs_layout_passes=False, use_tc_tiling_on_sc=False),
)
def _pool(ha_hbm, hc_hbm, hm_hbm, b_hbm,
          sa_hbm, sc_hbm, sm_hbm, xa_hbm, xc_hbm, xm_hbm,
          ha_v, hc_v, hm_v, b_v, sa_v, sc_v, sm_v, xa_v, xc_v, xm_v):
    c = lax.axis_index("c")
    s = lax.axis_index("s")
    w = c * NS + s
    pltpu.sync_copy(ha_hbm.at[pl.ds(w * NPW, NPW)], ha_v)
    pltpu.sync_copy(hc_hbm.at[pl.ds(w * NPW, NPW)], hc_v)
    pltpu.sync_copy(hm_hbm.at[pl.ds(w * NPW, NPW)], hm_v)
    pltpu.sync_copy(b_hbm.at[w], b_v)

    def init_row(i, carry):
        zv = jnp.zeros((16,), jnp.float32)
        nv = jnp.full((16,), _NEG, jnp.float32)
        sa_v[i, :] = zv
        sc_v[i, :] = zv
        sm_v[i, :] = zv
        xa_v[i, :] = nv
        xc_v[i, :] = nv
        xm_v[i, :] = nv
        return carry
    lax.fori_loop(0, GP, init_row, 0)

    iota16 = lax.broadcasted_iota(jnp.int32, (16,), 0)

    def scan_node(i, carry):
        cur, s0, s1, s2, m0, m1, m2 = carry
        g = i // 16
        l = i - g * 16
        bvec = b_v[g, :]
        seg = jnp.sum(jnp.where(iota16 == l, bvec, 0))
        r0 = ha_v[i, :]
        r1 = hc_v[i, :]
        r2 = hm_v[i, :]
        changed = seg != cur
        s0 = jnp.where(changed, r0, s0 + r0)
        s1 = jnp.where(changed, r1, s1 + r1)
        s2 = jnp.where(changed, r2, s2 + r2)
        m0 = jnp.where(changed, r0, jnp.maximum(m0, r0))
        m1 = jnp.where(changed, r1, jnp.maximum(m1, r1))
        m2 = jnp.where(changed, r2, jnp.maximum(m2, r2))
        sa_v[seg, :] = s0
        sc_v[seg, :] = s1
        sm_v[seg, :] = s2
        xa_v[seg, :] = m0
        xc_v[seg, :] = m1
        xm_v[seg, :] = m2
        return (seg, s0, s1, s2, m0, m1, m2)

    z = jnp.zeros((16,), jnp.float32)
    ninf = jnp.full((16,), _NEG, jnp.float32)
    lax.fori_loop(
        0, NPW, scan_node,
        (jnp.int32(-1), z, z, z, ninf, ninf, ninf))

    pltpu.sync_copy(sa_v, sa_hbm.at[w])
    pltpu.sync_copy(sc_v, sc_hbm.at[w])
    pltpu.sync_copy(sm_v, sm_hbm.at[w])
    pltpu.sync_copy(xa_v, xa_hbm.at[w])
    pltpu.sync_copy(xc_v, xc_hbm.at[w])
    pltpu.sync_copy(xm_v, xm_hbm.at[w])


def kernel(x, edge_index, batch,
           Wa0, bWa0, Wc0, bWc0, Wm10, bWm10, Wm20, bWm20, g0, beta0,
           Wa1, bWa1, Wc1, bWc1, Wm11, bWm11, Wm21, bWm21, g1, beta1,
           Wa2, bWa2, Wc2, bWc2, Wm12, bWm12, Wm22, bWm22, g2, beta2,
           Wa3, bWa3, Wc3, bWc3, Wm13, bWm13, Wm23, bWm23, g3, beta3,
           W2, b2):
    inv = 1.0 / jnp.sqrt(jnp.asarray(1.0 + EPS, jnp.float32))
    Wams, Wcs, pps, cps = [], [], [], []
    zero16 = jnp.zeros((16,), jnp.float32)
    for (Wa, bWa, Wc, bWc, Wm1, bWm1, Wm2, bWm2, g, beta) in (
            (Wa0, bWa0, Wc0, bWc0, Wm10, bWm10, Wm20, bWm20, g0, beta0),
            (Wa1, bWa1, Wc1, bWc1, Wm11, bWm11, Wm21, bWm21, g1, beta1),
            (Wa2, bWa2, Wc2, bWc2, Wm12, bWm12, Wm22, bWm22, g2, beta2),
            (Wa3, bWa3, Wc3, bWc3, Wm13, bWm13, Wm23, bWm23, g3, beta3)):
        sca = g * inv
        Wams.append(jnp.concatenate([Wa, Wm1, Wm2], axis=1))
        Wcs.append(Wc)
        pps.append(jnp.stack([bWa, bWm1, bWm2, sca[0:16], beta[0:16],
                              sca[32:48], beta[32:48], zero16]))
        cps.append(jnp.stack([bWc, sca[16:32], beta[16:32], zero16,
                              zero16, zero16, zero16, zero16]))
    W0 = jnp.concatenate([Wams[0], Wcs[0]], axis=1)

    src = edge_index[0]
    dst = edge_index[1]
    pad = E_PAD - E
    srcp = jnp.concatenate([src, jnp.zeros((pad,), jnp.int32)]
                           ).reshape(NW, CH, 1, CHUNK)
    dstp = jnp.concatenate([dst, jnp.full((pad,), N, jnp.int32)]
                           ).reshape(NW, CH, 1, CHUNK)
    batch_pad = jnp.concatenate(
        [batch, jnp.full((N_POOL - N,), G, jnp.int32)]
    ).reshape(NW, NPW // 16, 16)
    batch3d = batch.reshape(GRID, 1, BN)
    w2b = jnp.zeros((104, 6), jnp.float32).at[:96].set(W2).at[96].set(b2)

    a0, m0, hc0 = _d0(x, W0, pps[0])
    P0 = _agg(hc0, srcp, dstp)
    c0, a1, m1 = _d1(a0, m0, P0, cps[0], Wams[1], pps[1])
    qa, qc, qm = _agg3(a0, c0, m0, srcp, dstp)
    c1, a2, m2 = _dq(a1, m1, qa, qc, qm, Wcs[1], cps[1], Wams[2], pps[2])
    qa, qc, qm = _agg3(a1, c1, m1, srcp, dstp)
    c2, a3, m3 = _dq(a2, m2, qa, qc, qm, Wcs[2], cps[2], Wams[3], pps[3])
    qa, qc, qm = _agg3(a2, c2, m2, srcp, dstp)
    c3 = _d4(qa, qc, qm, Wcs[3], cps[3])
    sa, sc, sm, xa, xc, xm = _pool(a3, c3, m3, batch_pad)
    return _d5(batch3d, sa, sc, sm, xa, xc, xm, w2b)


# speedup vs baseline: 11.8804x; 2.0405x over previous
import functools

import jax
import jax.numpy as jnp
from jax import lax
from jax.experimental import pallas as pl
from jax.experimental.pallas import tpu as pltpu
from jax.experimental.pallas import tpu_sc as plsc

N = 10000
E = 320000
F_IN = 128
NOUT = 16
NIN = 48
G = 128
EPS = 1e-5

NC = 2
NS = 16
NW = NC * NS
CHUNK = 128
CH = 80
EPW = CH * CHUNK
E_PAD = NW * EPW
RPS = 632
N_ACC = NS * RPS
NPW = 320
N_POOL = NW * NPW
GP = 136

BN = 1000
GRID = N // BN


def _proj(z, pp):
    za, zm1, zm2 = z[:, 0:16], z[:, 16:32], z[:, 32:48]
    a = pp[3:4] * jax.nn.relu(za + pp[0:1]) + pp[4:5]
    m = pp[5:6] * (jax.nn.relu(zm1 + pp[1:2]) * jax.nn.relu(zm2 + pp[2:3])) + pp[6:7]
    return jnp.concatenate([a, m], axis=1)


def _d0_body(x_ref, w_ref, pp_ref, am_ref, hc_ref):
    z = jnp.dot(x_ref[...], w_ref[...], preferred_element_type=jnp.float32)
    am_ref[...] = _proj(z, pp_ref[...])
    hc_ref[...] = z[:, 48:64]


def _finish_c(p, cp):
    return cp[1:2] * jax.nn.relu(p[0] + p[1] + cp[0:1]) + cp[2:3]


def _dmid_body(am_ref, p_ref, cp_ref, w_ref, pp_ref, am_o, hc_o):
    amp = am_ref[...]
    c = _finish_c(p_ref[...], cp_ref[...])
    h = jnp.concatenate([amp[:, 0:16], c, amp[:, 16:32]], axis=1)
    z = jnp.dot(h, w_ref[...], preferred_element_type=jnp.float32)
    am_o[...] = _proj(z, pp_ref[...])
    hc_o[...] = z[:, 48:64]


def _d4_body(am_ref, p_ref, cp_ref, ha_o, hc_o, hm_o):
    amp = am_ref[...]
    ha_o[...] = amp[:, 0:16]
    hc_o[...] = _finish_c(p_ref[...], cp_ref[...])
    hm_o[...] = amp[:, 16:32]


def _d5_body(b_ref, sa_ref, sc_ref, sm_ref, xa_ref, xc_ref, xm_ref,
             w2_ref, out_ref):
    b = b_ref[...]
    iota = lax.broadcasted_iota(jnp.int32, (GRID, G, BN), 1)
    mask = (b == iota).astype(jnp.float32)
    m2 = jnp.sum(mask, axis=0)
    counts = jnp.dot(m2, jnp.ones((BN, 1), jnp.float32),
                     preferred_element_type=jnp.float32)
    sums = jnp.concatenate(
        [jnp.sum(r[...][:, :G, :], axis=0) for r in (sa_ref, sc_ref, sm_ref)],
        axis=1)
    maxs = jnp.concatenate(
        [jnp.max(r[...][:, :G, :], axis=0) for r in (xa_ref, xc_ref, xm_ref)],
        axis=1)
    meanp = sums / jnp.maximum(counts, 1.0)
    hg = jnp.concatenate([meanp, maxs], axis=1)
    w2b = w2_ref[...]
    logits = jnp.dot(hg, w2b[:96], preferred_element_type=jnp.float32) + w2b[96:97]
    mx = jnp.max(logits, axis=1, keepdims=True)
    sh = logits - mx
    out_ref[...] = sh - jnp.log(jnp.sum(jnp.exp(sh), axis=1, keepdims=True))


_d0 = pl.pallas_call(
    _d0_body,
    grid=(GRID,),
    in_specs=[
        pl.BlockSpec((BN, F_IN), lambda i: (i, 0)),
        pl.BlockSpec((F_IN, 64), lambda i: (0, 0)),
        pl.BlockSpec((8, 16), lambda i: (0, 0)),
    ],
    out_specs=[
        pl.BlockSpec((BN, 32), lambda i: (i, 0)),
        pl.BlockSpec((BN, 16), lambda i: (i, 0)),
    ],
    out_shape=[
        jax.ShapeDtypeStruct((N, 32), jnp.float32),
        jax.ShapeDtypeStruct((N_ACC, 16), jnp.float32),
    ],
)

_dmid = pl.pallas_call(
    _dmid_body,
    grid=(GRID,),
    in_specs=[
        pl.BlockSpec((BN, 32), lambda i: (i, 0)),
        pl.BlockSpec((NC, BN, 16), lambda i: (0, i, 0)),
        pl.BlockSpec((8, 16), lambda i: (0, 0)),
        pl.BlockSpec((NIN, 64), lambda i: (0, 0)),
        pl.BlockSpec((8, 16), lambda i: (0, 0)),
    ],
    out_specs=[
        pl.BlockSpec((BN, 32), lambda i: (i, 0)),
        pl.BlockSpec((BN, 16), lambda i: (i, 0)),
    ],
    out_shape=[
        jax.ShapeDtypeStruct((N, 32), jnp.float32),
        jax.ShapeDtypeStruct((N_ACC, 16), jnp.float32),
    ],
)

_d4 = pl.pallas_call(
    _d4_body,
    grid=(GRID,),
    in_specs=[
        pl.BlockSpec((BN, 32), lambda i: (i, 0)),
        pl.BlockSpec((NC, BN, 16), lambda i: (0, i, 0)),
        pl.BlockSpec((8, 16), lambda i: (0, 0)),
    ],
    out_specs=[
        pl.BlockSpec((BN, 16), lambda i: (i, 0)),
        pl.BlockSpec((BN, 16), lambda i: (i, 0)),
        pl.BlockSpec((BN, 16), lambda i: (i, 0)),
    ],
    out_shape=[
        jax.ShapeDtypeStruct((N_POOL, 16), jnp.float32),
        jax.ShapeDtypeStruct((N_POOL, 16), jnp.float32),
        jax.ShapeDtypeStruct((N_POOL, 16), jnp.float32),
    ],
)

_d5 = pl.pallas_call(
    _d5_body,
    grid=(1,),
    in_specs=[pl.BlockSpec((GRID, 1, BN), lambda i: (0, 0, 0))]
    + [pl.BlockSpec((NW, GP, 16), lambda i: (0, 0, 0))] * 6
    + [pl.BlockSpec((104, 6), lambda i: (0, 0))],
    out_specs=pl.BlockSpec((G, 6), lambda i: (0, 0)),
    out_shape=jax.ShapeDtypeStruct((G, 6), jnp.float32),
)


_MESH = plsc.VectorSubcoreMesh(core_axis_name="c", subcore_axis_name="s")


ZR = 79


@functools.partial(
    pl.kernel,
    out_type=jax.ShapeDtypeStruct((NC, N_ACC, NOUT), jnp.float32),
    mesh=_MESH,
    scratch_types=[
        pltpu.VMEM((2, 1, CHUNK), jnp.int32),
        pltpu.VMEM((2, 1, CHUNK), jnp.int32),
        pltpu.VMEM((2, CHUNK, NOUT), jnp.float32),
        pltpu.VMEM((ZR, NOUT), jnp.float32),
        pltpu.VMEM_SHARED((N_ACC, NOUT), jnp.float32),
        pltpu.SemaphoreType.DMA,
        pltpu.SemaphoreType.DMA,
        pltpu.SemaphoreType.DMA,
        pltpu.SemaphoreType.DMA,
    ],
    compiler_params=pltpu.CompilerParams(use_tc_tiling_on_sc=False),
)
def _agg(table_hbm, src_hbm, dst_hbm, out_hbm,
         src_v, dst_v, rows_v, zbuf_v, acc_sh, g0, g1, i0, i1):
    c = lax.axis_index("c")
    s = lax.axis_index("s")
    w = c * NS + s
    gsem = (g0, g1)
    isem = (i0, i1)

    def zero_row(i, carry):
        zbuf_v[i, :] = jnp.zeros((NOUT,), jnp.float32)
        return carry
    lax.fori_loop(0, ZR, zero_row, 0)
    for k in range(RPS // ZR):
        pltpu.sync_copy(zbuf_v, acc_sh.at[pl.ds(s * RPS + k * ZR, ZR)])
    plsc.subcore_barrier()

    def idx_start(j, p, sem):
        pltpu.make_async_copy(src_hbm.at[w, j], src_v.at[p], sem).start()
        pltpu.make_async_copy(dst_hbm.at[w, j], dst_v.at[p], sem).start()

    def idx_wait(p, sem):
        pltpu.make_async_copy(src_hbm.at[w, 0], src_v.at[p], sem).wait()
        pltpu.make_async_copy(dst_hbm.at[w, 0], dst_v.at[p], sem).wait()

    def gather_start(p, sem):
        pltpu.make_async_copy(
            table_hbm.at[src_v.at[p, 0]], rows_v.at[p], sem).start()

    def gather_wait(p, sem):
        pltpu.make_async_copy(
            table_hbm.at[src_v.at[p, 0]], rows_v.at[p], sem).wait()

    idx_start(0, 0, isem[0])
    idx_wait(0, isem[0])
    gather_start(0, gsem[0])
    idx_start(1, 1, isem[1])

    def half_step(j, p):
        q = 1 - p
        idx_wait(q, isem[q])
        gather_start(q, gsem[q])
        gather_wait(p, gsem[p])
        pltpu.sync_copy(rows_v.at[p], acc_sh.at[dst_v.at[p, 0]], add=True)
        nj = j + 2
        nj = jnp.where(nj >= CH, nj - CH, nj)
        idx_start(nj, p, isem[p])

    def pair(jj, carry):
        half_step(2 * jj, 0)
        half_step(2 * jj + 1, 1)
        return carry
    lax.fori_loop(0, CH // 2, pair, 0)
    gather_wait(0, gsem[0])
    idx_wait(1, isem[1])
    plsc.subcore_barrier()
    pltpu.sync_copy(acc_sh.at[pl.ds(s * RPS, RPS)],
                    out_hbm.at[c, pl.ds(s * RPS, RPS)])


_NEG = float("-inf")


@functools.partial(
    pl.kernel,
    out_type=[jax.ShapeDtypeStruct((NW, GP, 16), jnp.float32)] * 6,
    mesh=_MESH,
    scratch_types=[
        pltpu.VMEM((NPW, 16), jnp.float32),
        pltpu.VMEM((NPW, 16), jnp.float32),
        pltpu.VMEM((NPW, 16), jnp.float32),
        pltpu.VMEM((NPW // 16, 16), jnp.int32),
        pltpu.VMEM((GP, 16), jnp.float32),
        pltpu.VMEM((GP, 16), jnp.float32),
        pltpu.VMEM((GP, 16), jnp.float32),
        pltpu.VMEM((GP, 16), jnp.float32),
        pltpu.VMEM((GP, 16), jnp.float32),
        pltpu.VMEM((GP, 16), jnp.float32),
    ],
    compiler_params=pltpu.CompilerParams(
        needs_layout_passes=False, use_tc_tiling_on_sc=False),
)
def _pool(ha_hbm, hc_hbm, hm_hbm, b_hbm,
          sa_hbm, sc_hbm, sm_hbm, xa_hbm, xc_hbm, xm_hbm,
          ha_v, hc_v, hm_v, b_v, sa_v, sc_v, sm_v, xa_v, xc_v, xm_v):
    c = lax.axis_index("c")
    s = lax.axis_index("s")
    w = c * NS + s
    pltpu.sync_copy(ha_hbm.at[pl.ds(w * NPW, NPW)], ha_v)
    pltpu.sync_copy(hc_hbm.at[pl.ds(w * NPW, NPW)], hc_v)
    pltpu.sync_copy(hm_hbm.at[pl.ds(w * NPW, NPW)], hm_v)
    pltpu.sync_copy(b_hbm.at[w], b_v)

    def init_row(i, carry):
        zv = jnp.zeros((16,), jnp.float32)
        nv = jnp.full((16,), _NEG, jnp.float32)
        sa_v[i, :] = zv
        sc_v[i, :] = zv
        sm_v[i, :] = zv
        xa_v[i, :] = nv
        xc_v[i, :] = nv
        xm_v[i, :] = nv
        return carry
    lax.fori_loop(0, GP, init_row, 0)

    iota16 = lax.broadcasted_iota(jnp.int32, (16,), 0)

    def scan_node(i, carry):
        cur, s0, s1, s2, m0, m1, m2 = carry
        g = i // 16
        l = i - g * 16
        bvec = b_v[g, :]
        seg = jnp.sum(jnp.where(iota16 == l, bvec, 0))
        r0 = ha_v[i, :]
        r1 = hc_v[i, :]
        r2 = hm_v[i, :]
        changed = seg != cur
        s0 = jnp.where(changed, r0, s0 + r0)
        s1 = jnp.where(changed, r1, s1 + r1)
        s2 = jnp.where(changed, r2, s2 + r2)
        m0 = jnp.where(changed, r0, jnp.maximum(m0, r0))
        m1 = jnp.where(changed, r1, jnp.maximum(m1, r1))
        m2 = jnp.where(changed, r2, jnp.maximum(m2, r2))
        sa_v[seg, :] = s0
        sc_v[seg, :] = s1
        sm_v[seg, :] = s2
        xa_v[seg, :] = m0
        xc_v[seg, :] = m1
        xm_v[seg, :] = m2
        return (seg, s0, s1, s2, m0, m1, m2)

    z = jnp.zeros((16,), jnp.float32)
    ninf = jnp.full((16,), _NEG, jnp.float32)
    lax.fori_loop(
        0, NPW, scan_node,
        (jnp.int32(-1), z, z, z, ninf, ninf, ninf))

    pltpu.sync_copy(sa_v, sa_hbm.at[w])
    pltpu.sync_copy(sc_v, sc_hbm.at[w])
    pltpu.sync_copy(sm_v, sm_hbm.at[w])
    pltpu.sync_copy(xa_v, xa_hbm.at[w])
    pltpu.sync_copy(xc_v, xc_hbm.at[w])
    pltpu.sync_copy(xm_v, xm_hbm.at[w])


def kernel(x, edge_index, batch,
           Wa0, bWa0, Wc0, bWc0, Wm10, bWm10, Wm20, bWm20, g0, beta0,
           Wa1, bWa1, Wc1, bWc1, Wm11, bWm11, Wm21, bWm21, g1, beta1,
           Wa2, bWa2, Wc2, bWc2, Wm12, bWm12, Wm22, bWm22, g2, beta2,
           Wa3, bWa3, Wc3, bWc3, Wm13, bWm13, Wm23, bWm23, g3, beta3,
           W2, b2):
    inv = 1.0 / jnp.sqrt(jnp.asarray(1.0 + EPS, jnp.float32))
    Ws, pps, cps = [], [], []
    zero16 = jnp.zeros((16,), jnp.float32)
    for (Wa, bWa, Wc, bWc, Wm1, bWm1, Wm2, bWm2, g, beta) in (
            (Wa0, bWa0, Wc0, bWc0, Wm10, bWm10, Wm20, bWm20, g0, beta0),
            (Wa1, bWa1, Wc1, bWc1, Wm11, bWm11, Wm21, bWm21, g1, beta1),
            (Wa2, bWa2, Wc2, bWc2, Wm12, bWm12, Wm22, bWm22, g2, beta2),
            (Wa3, bWa3, Wc3, bWc3, Wm13, bWm13, Wm23, bWm23, g3, beta3)):
        sca = g * inv
        Ws.append(jnp.concatenate([Wa, Wm1, Wm2, Wc], axis=1))
        pps.append(jnp.stack([bWa, bWm1, bWm2, sca[0:16], beta[0:16],
                              sca[32:48], beta[32:48], zero16]))
        cps.append(jnp.stack([bWc, sca[16:32], beta[16:32], zero16,
                              zero16, zero16, zero16, zero16]))

    src = edge_index[0]
    dst = edge_index[1]
    pad = E_PAD - E
    srcp = jnp.concatenate([src, jnp.zeros((pad,), jnp.int32)]
                           ).reshape(NW, CH, 1, CHUNK)
    dstp = jnp.concatenate([dst, jnp.full((pad,), N, jnp.int32)]
                           ).reshape(NW, CH, 1, CHUNK)
    batch_pad = jnp.concatenate(
        [batch, jnp.full((N_POOL - N,), G, jnp.int32)]
    ).reshape(NW, NPW // 16, 16)
    batch3d = batch.reshape(GRID, 1, BN)
    w2b = jnp.zeros((104, 6), jnp.float32).at[:96].set(W2).at[96].set(b2)

    am, hc = _d0(x, Ws[0], pps[0])
    P = _agg(hc, srcp, dstp)
    for i in (1, 2, 3):
        am, hc = _dmid(am, P, cps[i - 1], Ws[i], pps[i])
        P = _agg(hc, srcp, dstp)
    ha, hcp, hm = _d4(am, P, cps[3])
    sa, sc, sm, xa, xc, xm = _pool(ha, hcp, hm, batch_pad)
    return _d5(batch3d, sa, sc, sm, xa, xc, xm, w2b)
